# Initial kernel scaffold; baseline (speedup 1.0000x reference)
import jax
import jax.numpy as jnp
from jax.experimental import pallas as pl


def kernel(boxes, scores):
    raise NotImplementedError("write your pallas kernel here")


# speedup vs baseline: 8.2642x; 8.2642x over previous
import functools

import jax
import jax.numpy as jnp
from jax.experimental import pallas as pl
from jax.experimental.pallas import tpu as pltpu

_N = 20000
_NPAD = 20480
_K = 4096
_THR = 0.5
_JCH = 2048
_CH = 256


def _rank_kernel(s_col_ref, s_row_ref, rank_ref):
    i0 = pl.program_id(0) * 256
    si = s_col_ref[:, :]
    ii = i0 + jax.lax.broadcasted_iota(jnp.int32, (256, 1), 0)
    acc = jnp.zeros((256, 1), jnp.int32)
    for c in range(_NPAD // _JCH):
        sj = s_row_ref[:, c * _JCH:(c + 1) * _JCH]
        jj = c * _JCH + jax.lax.broadcasted_iota(jnp.int32, (1, _JCH), 1)
        beats = (sj > si) | ((sj == si) & (jj < ii))
        acc = acc + jnp.sum(beats.astype(jnp.int32), axis=1, keepdims=True)
    rank_ref[:, :] = acc


def _select_kernel(rank_row_ref, data_ref, out_ref):
    p0 = pl.program_id(0) * 512
    pp = p0 + jax.lax.broadcasted_iota(jnp.int32, (512, 1), 0)
    acc = jnp.zeros((512, 8), jnp.float32)
    for c in range(_NPAD // _JCH):
        rr = rank_row_ref[:, c * _JCH:(c + 1) * _JCH]
        oh = (rr == pp).astype(jnp.float32)
        d = data_ref[c * _JCH:(c + 1) * _JCH, :]
        acc = acc + jax.lax.dot_general(
            oh, d, (((1,), (0,)), ((), ())),
            preferred_element_type=jnp.float32,
            precision=jax.lax.Precision.HIGHEST)
    out_ref[:, :] = acc


def _nms_kernel(top_ref, top_t_ref, out_ref, s_scr, kc_scr):
    x1r = top_t_ref[0:1, :]
    y1r = top_t_ref[1:2, :]
    x2r = top_t_ref[2:3, :]
    y2r = top_t_ref[3:4, :]
    area_r = (x2r - x1r) * (y2r - y1r)
    jj_row = jax.lax.broadcasted_iota(jnp.int32, (_CH, _K), 1)

    def s_body(ci, _):
        sl = pl.ds(ci * _CH, _CH)
        x1c = top_ref[sl, 0:1]
        y1c = top_ref[sl, 1:2]
        x2c = top_ref[sl, 2:3]
        y2c = top_ref[sl, 3:4]
        ix1 = jnp.maximum(x1c, x1r)
        iy1 = jnp.maximum(y1c, y1r)
        ix2 = jnp.minimum(x2c, x2r)
        iy2 = jnp.minimum(y2c, y2r)
        iw = jnp.maximum(ix2 - ix1, 0.0)
        ih = jnp.maximum(iy2 - iy1, 0.0)
        inter = iw * ih
        union = (x2c - x1c) * (y2c - y1c) + area_r - inter
        iou = inter / jnp.maximum(union, 1e-9)
        s_scr[sl, :] = (iou > _THR).astype(jnp.int8)
        return 0

    jax.lax.fori_loop(0, _K // _CH, s_body, 0)
    kc_scr[:, :] = jnp.ones((_K, 1), jnp.float32)

    def t_row_body(ci, supp):
        sl = pl.ds(ci * _CH, _CH)
        sb = s_scr[sl, :] != 0
        ii = ci * _CH + jax.lax.broadcasted_iota(jnp.int32, (_CH, _K), 0)
        kc = kc_scr[sl, :] > 0.0
        hit = (sb & (ii < jj_row) & kc).astype(jnp.float32)
        return jnp.maximum(supp, jnp.max(hit, axis=0, keepdims=True))

    def t_col_body(cj, keep_r):
        sl = pl.ds(cj * _CH, _CH)
        sb = s_scr[sl, :] != 0
        jj = cj * _CH + jax.lax.broadcasted_iota(jnp.int32, (_CH, _K), 0)
        hit = (sb & (jj_row < jj) & (keep_r > 0.0)).astype(jnp.float32)
        kc_scr[sl, :] = 1.0 - jnp.max(hit, axis=1, keepdims=True)
        return keep_r

    def cond(st):
        done, it = st
        return jnp.logical_and(jnp.logical_not(done), it < _K)

    def body(st):
        _, it = st
        kc_old = kc_scr[:, :]
        supp = jax.lax.fori_loop(
            0, _K // _CH, t_row_body, jnp.zeros((1, _K), jnp.float32))
        keep_r = 1.0 - supp
        jax.lax.fori_loop(0, _K // _CH, t_col_body, keep_r)
        done = jnp.all(kc_scr[:, :] == kc_old)
        return done, it + 1

    jax.lax.while_loop(cond, body, (False, jnp.int32(0)))
    out_ref[:, :] = top_ref[:, :] * kc_scr[:, :]


@jax.jit
def kernel(boxes, scores):
    f32 = jnp.float32
    s_pad = jnp.concatenate(
        [scores.astype(f32), jnp.full((_NPAD - _N,), -1.0, f32)])
    s_col = s_pad.reshape(_NPAD, 1)
    s_row = s_pad.reshape(1, _NPAD)

    ranks = pl.pallas_call(
        _rank_kernel,
        grid=(_NPAD // 256,),
        in_specs=[
            pl.BlockSpec((256, 1), lambda i: (i, 0)),
            pl.BlockSpec((1, _NPAD), lambda i: (0, 0)),
        ],
        out_specs=pl.BlockSpec((256, 1), lambda i: (i, 0)),
        out_shape=jax.ShapeDtypeStruct((_NPAD, 1), jnp.int32),
    )(s_col, s_row)

    data = jnp.concatenate(
        [boxes.astype(f32), scores.astype(f32)[:, None],
         jnp.zeros((_N, 3), f32)], axis=1)
    data = jnp.concatenate([data, jnp.zeros((_NPAD - _N, 8), f32)], axis=0)
    rank_row = ranks.reshape(1, _NPAD)

    top = pl.pallas_call(
        _select_kernel,
        grid=(_K // 512,),
        in_specs=[
            pl.BlockSpec((1, _NPAD), lambda i: (0, 0)),
            pl.BlockSpec((_NPAD, 8), lambda i: (0, 0)),
        ],
        out_specs=pl.BlockSpec((512, 8), lambda i: (i, 0)),
        out_shape=jax.ShapeDtypeStruct((_K, 8), f32),
    )(rank_row, data)

    out = pl.pallas_call(
        _nms_kernel,
        in_specs=[
            pl.BlockSpec((_K, 8), lambda: (0, 0)),
            pl.BlockSpec((8, _K), lambda: (0, 0)),
        ],
        out_specs=pl.BlockSpec((_K, 8), lambda: (0, 0)),
        out_shape=jax.ShapeDtypeStruct((_K, 8), f32),
        scratch_shapes=[pltpu.VMEM((_K, _K), jnp.int8),
                        pltpu.VMEM((_K, 1), jnp.float32)],
    )(top, top.T)
    return out[:, :5]


# speedup vs baseline: 9.9258x; 1.2010x over previous
import functools

import jax
import jax.numpy as jnp
from jax.experimental import pallas as pl
from jax.experimental.pallas import tpu as pltpu

_N = 20000
_NPAD = 20480
_K = 4096
_THR = 0.5
_JCH = 2048
_CH = 256


def _rank_kernel(s_col_ref, s_row_ref, rank_ref):
    i0 = pl.program_id(0) * 256
    si = s_col_ref[:, :]
    ii = i0 + jax.lax.broadcasted_iota(jnp.int32, (256, 1), 0)
    acc = jnp.zeros((256, 1), jnp.int32)
    for c in range(_NPAD // _JCH):
        sj = s_row_ref[:, c * _JCH:(c + 1) * _JCH]
        jj = c * _JCH + jax.lax.broadcasted_iota(jnp.int32, (1, _JCH), 1)
        beats = (sj > si) | ((sj == si) & (jj < ii))
        acc = acc + jnp.sum(beats.astype(jnp.int32), axis=1, keepdims=True)
    rank_ref[:, :] = acc


def _select_kernel(rank_row_ref, data_ref, out_ref):
    p0 = pl.program_id(0) * 512
    pp = p0 + jax.lax.broadcasted_iota(jnp.int32, (512, 1), 0)
    acc = jnp.zeros((512, 8), jnp.float32)
    for c in range(_NPAD // _JCH):
        rr = rank_row_ref[:, c * _JCH:(c + 1) * _JCH]
        oh = (rr == pp).astype(jnp.float32)
        d = data_ref[c * _JCH:(c + 1) * _JCH, :]
        acc = acc + jax.lax.dot_general(
            oh, d, (((1,), (0,)), ((), ())),
            preferred_element_type=jnp.float32,
            precision=jax.lax.Precision.HIGHEST)
    out_ref[:, :] = acc


_B = 512


def _nms_kernel(top_ref, top_t_ref, out_ref, s_scr, kc_scr, kr_scr):
    x1r = top_t_ref[0:1, :]
    y1r = top_t_ref[1:2, :]
    x2r = top_t_ref[2:3, :]
    y2r = top_t_ref[3:4, :]
    area_r = (x2r - x1r) * (y2r - y1r)

    def s_body(ci, _):
        sl = pl.ds(ci * _CH, _CH)
        x1c = top_ref[sl, 0:1]
        y1c = top_ref[sl, 1:2]
        x2c = top_ref[sl, 2:3]
        y2c = top_ref[sl, 3:4]
        ix1 = jnp.maximum(x1c, x1r)
        iy1 = jnp.maximum(y1c, y1r)
        ix2 = jnp.minimum(x2c, x2r)
        iy2 = jnp.minimum(y2c, y2r)
        iw = jnp.maximum(ix2 - ix1, 0.0)
        ih = jnp.maximum(iy2 - iy1, 0.0)
        inter = iw * ih
        union = (x2c - x1c) * (y2c - y1c) + area_r - inter
        iou = inter / jnp.maximum(union, 1e-9)
        s_scr[sl, :] = (iou > _THR).astype(jnp.int8)
        return 0

    jax.lax.fori_loop(0, _K // _CH, s_body, 0)
    kr_scr[:, :] = jnp.ones((1, _K), jnp.float32)

    rr = jax.lax.broadcasted_iota(jnp.int32, (_B, _B), 0)
    cc = jax.lax.broadcasted_iota(jnp.int32, (_B, _B), 1)
    ident = (rr == cc).astype(jnp.float32)
    tri_up = rr < cc
    tri_lo = cc < rr
    jlane = jax.lax.broadcasted_iota(jnp.int32, (1, _K), 1)

    for b in range(_K // _B):
        lo = b * _B
        hi = (b + 1) * _B
        sbb = s_scr[lo:hi, lo:hi] != 0
        up_m = sbb & tri_up
        lo_m = sbb & tri_lo
        ext_row = kr_scr[0:1, lo:hi]
        ext_col = jnp.sum(ident * ext_row, axis=1, keepdims=True)

        def cond(st):
            _, done, it = st
            return jnp.logical_and(jnp.logical_not(done), it < _B)

        def body(st, ext_row=ext_row, ext_col=ext_col, up_m=up_m, lo_m=lo_m):
            kcol, _, it = st
            hit_r = jnp.max((up_m & (kcol > 0.0)).astype(jnp.float32),
                            axis=0, keepdims=True)
            krow = ext_row * (1.0 - hit_r)
            hit_c = jnp.max((lo_m & (krow > 0.0)).astype(jnp.float32),
                            axis=1, keepdims=True)
            kcol2 = ext_col * (1.0 - hit_c)
            done = jnp.all(kcol2 == kcol)
            return kcol2, done, it + 1

        kcol, _, _ = jax.lax.while_loop(
            cond, body, (ext_col, False, jnp.int32(0)))
        kc_scr[lo:hi, :] = kcol
        if b + 1 < _K // _B:
            srow = s_scr[lo:hi, :] != 0
            supp = jnp.max((srow & (kcol > 0.0)).astype(jnp.float32),
                           axis=0, keepdims=True)
            later = (jlane >= hi).astype(jnp.float32)
            kr_scr[0:1, :] = kr_scr[0:1, :] * (1.0 - supp * later)

    out_ref[:, :] = top_ref[:, :] * kc_scr[:, :]


@jax.jit
def kernel(boxes, scores):
    f32 = jnp.float32
    s_pad = jnp.concatenate(
        [scores.astype(f32), jnp.full((_NPAD - _N,), -1.0, f32)])
    s_col = s_pad.reshape(_NPAD, 1)
    s_row = s_pad.reshape(1, _NPAD)

    ranks = pl.pallas_call(
        _rank_kernel,
        grid=(_NPAD // 256,),
        in_specs=[
            pl.BlockSpec((256, 1), lambda i: (i, 0)),
            pl.BlockSpec((1, _NPAD), lambda i: (0, 0)),
        ],
        out_specs=pl.BlockSpec((256, 1), lambda i: (i, 0)),
        out_shape=jax.ShapeDtypeStruct((_NPAD, 1), jnp.int32),
    )(s_col, s_row)

    data = jnp.concatenate(
        [boxes.astype(f32), scores.astype(f32)[:, None],
         jnp.zeros((_N, 3), f32)], axis=1)
    data = jnp.concatenate([data, jnp.zeros((_NPAD - _N, 8), f32)], axis=0)
    rank_row = ranks.reshape(1, _NPAD)

    top = pl.pallas_call(
        _select_kernel,
        grid=(_K // 512,),
        in_specs=[
            pl.BlockSpec((1, _NPAD), lambda i: (0, 0)),
            pl.BlockSpec((_NPAD, 8), lambda i: (0, 0)),
        ],
        out_specs=pl.BlockSpec((512, 8), lambda i: (i, 0)),
        out_shape=jax.ShapeDtypeStruct((_K, 8), f32),
    )(rank_row, data)

    out = pl.pallas_call(
        _nms_kernel,
        in_specs=[
            pl.BlockSpec((_K, 8), lambda: (0, 0)),
            pl.BlockSpec((8, _K), lambda: (0, 0)),
        ],
        out_specs=pl.BlockSpec((_K, 8), lambda: (0, 0)),
        out_shape=jax.ShapeDtypeStruct((_K, 8), f32),
        scratch_shapes=[pltpu.VMEM((_K, _K), jnp.int8),
                        pltpu.VMEM((_K, 1), jnp.float32),
                        pltpu.VMEM((1, _K), jnp.float32)],
    )(top, top.T)
    return out[:, :5]


# speedup vs baseline: 21.7098x; 2.1872x over previous
import functools

import jax
import jax.numpy as jnp
from jax.experimental import pallas as pl
from jax.experimental.pallas import tpu as pltpu

_N = 20000
_NPAD = 20480
_K = 4096
_THR = 0.5
_JCH = 2048
_CH = 256


def _sort_key(x):
    b = jax.lax.bitcast_convert_type(x, jnp.int32)
    return jnp.where(b >= 0, b, b ^ jnp.int32(0x7FFFFFFF))


def _rank_kernel(s_col_ref, s_row_ref, rank_ref):
    nblk_per_chunk = _JCH // 256
    i0 = pl.program_id(0) * 256
    ki = _sort_key(s_col_ref[:, :])
    ii = i0 + jax.lax.broadcasted_iota(jnp.int32, (256, 1), 0)
    cblk = pl.program_id(0) // nblk_per_chunk
    acc = jnp.zeros((256, 1), jnp.float32)
    for c in range(_NPAD // _JCH):
        kj = _sort_key(s_row_ref[:, c * _JCH:(c + 1) * _JCH])
        thr = jnp.where(c < cblk, ki - 1, ki)
        beats = kj > thr
        acc = acc + jnp.sum(beats.astype(jnp.float32), axis=1, keepdims=True)
    rank_ref[:, :] = acc.astype(jnp.int32)
    for c in range(_NPAD // _JCH):
        @pl.when(c == cblk)
        def _(c=c):
            kj = _sort_key(s_row_ref[:, c * _JCH:(c + 1) * _JCH])
            jj = c * _JCH + jax.lax.broadcasted_iota(jnp.int32, (1, _JCH), 1)
            tie = (kj == ki) & (jj < ii)
            corr = jnp.sum(tie.astype(jnp.float32), axis=1, keepdims=True)
            rank_ref[:, :] = rank_ref[:, :] + corr.astype(jnp.int32)


def _select_kernel(rank_row_ref, data_ref, out_ref):
    bf16 = jnp.bfloat16
    f32 = jnp.float32
    p0 = pl.program_id(0) * 512
    pp = p0 + jax.lax.broadcasted_iota(jnp.int32, (512, 1), 0)
    acc = jnp.zeros((512, 24), jnp.float32)
    for c in range(_NPAD // _JCH):
        rr = rank_row_ref[:, c * _JCH:(c + 1) * _JCH]
        oh = (rr == pp).astype(bf16)
        d = data_ref[c * _JCH:(c + 1) * _JCH, :]
        d_hi = d.astype(bf16)
        r1 = d - d_hi.astype(f32)
        d_mid = r1.astype(bf16)
        r2 = r1 - d_mid.astype(f32)
        d_lo = r2.astype(bf16)
        dcat = jnp.concatenate([d_hi, d_mid, d_lo], axis=1)
        acc = acc + jax.lax.dot_general(
            oh, dcat, (((1,), (0,)), ((), ())),
            preferred_element_type=jnp.float32)
    out_ref[:, :] = acc[:, 0:8] + acc[:, 8:16] + acc[:, 16:24]


_B = 512


def _nms_kernel(top_ref, top_t_ref, out_ref, s_scr, kc_scr, kr_scr):
    x1r = top_t_ref[0:1, :]
    y1r = top_t_ref[1:2, :]
    x2r = top_t_ref[2:3, :]
    y2r = top_t_ref[3:4, :]
    area_r = (x2r - x1r) * (y2r - y1r)

    def s_body(ci, _):
        sl = pl.ds(ci * _CH, _CH)
        x1c = top_ref[sl, 0:1]
        y1c = top_ref[sl, 1:2]
        x2c = top_ref[sl, 2:3]
        y2c = top_ref[sl, 3:4]
        ix1 = jnp.maximum(x1c, x1r)
        iy1 = jnp.maximum(y1c, y1r)
        ix2 = jnp.minimum(x2c, x2r)
        iy2 = jnp.minimum(y2c, y2r)
        iw = jnp.maximum(ix2 - ix1, 0.0)
        ih = jnp.maximum(iy2 - iy1, 0.0)
        inter = iw * ih
        union = (x2c - x1c) * (y2c - y1c) + area_r - inter
        iou = inter / jnp.maximum(union, 1e-9)
        s_scr[sl, :] = (iou > _THR).astype(jnp.int8)
        return 0

    jax.lax.fori_loop(0, _K // _CH, s_body, 0)
    kr_scr[:, :] = jnp.ones((1, _K), jnp.float32)

    rr = jax.lax.broadcasted_iota(jnp.int32, (_B, _B), 0)
    cc = jax.lax.broadcasted_iota(jnp.int32, (_B, _B), 1)
    ident = (rr == cc).astype(jnp.float32)
    tri_up = rr < cc
    tri_lo = cc < rr
    jlane = jax.lax.broadcasted_iota(jnp.int32, (1, _K), 1)

    for b in range(_K // _B):
        lo = b * _B
        hi = (b + 1) * _B
        sbb = s_scr[lo:hi, lo:hi] != 0
        up_m = sbb & tri_up
        lo_m = sbb & tri_lo
        ext_row = kr_scr[0:1, lo:hi]
        ext_col = jnp.sum(ident * ext_row, axis=1, keepdims=True)

        def cond(st):
            _, done, it = st
            return jnp.logical_and(jnp.logical_not(done), it < _B)

        def body(st, ext_row=ext_row, ext_col=ext_col, up_m=up_m, lo_m=lo_m):
            kcol, _, it = st
            hit_r = jnp.max((up_m & (kcol > 0.0)).astype(jnp.float32),
                            axis=0, keepdims=True)
            krow = ext_row * (1.0 - hit_r)
            hit_c = jnp.max((lo_m & (krow > 0.0)).astype(jnp.float32),
                            axis=1, keepdims=True)
            kcol2 = ext_col * (1.0 - hit_c)
            done = jnp.all(kcol2 == kcol)
            return kcol2, done, it + 1

        kcol, _, _ = jax.lax.while_loop(
            cond, body, (ext_col, False, jnp.int32(0)))
        kc_scr[lo:hi, :] = kcol
        if b + 1 < _K // _B:
            srow = s_scr[lo:hi, :] != 0
            supp = jnp.max((srow & (kcol > 0.0)).astype(jnp.float32),
                           axis=0, keepdims=True)
            later = (jlane >= hi).astype(jnp.float32)
            kr_scr[0:1, :] = kr_scr[0:1, :] * (1.0 - supp * later)

    out_ref[:, :] = top_ref[:, :] * kc_scr[:, :]


@jax.jit
def kernel(boxes, scores):
    f32 = jnp.float32
    s_pad = jnp.concatenate(
        [scores.astype(f32), jnp.full((_NPAD - _N,), -1.0, f32)])
    s_col = s_pad.reshape(_NPAD, 1)
    s_row = s_pad.reshape(1, _NPAD)

    ranks = pl.pallas_call(
        _rank_kernel,
        grid=(_NPAD // 256,),
        in_specs=[
            pl.BlockSpec((256, 1), lambda i: (i, 0)),
            pl.BlockSpec((1, _NPAD), lambda i: (0, 0)),
        ],
        out_specs=pl.BlockSpec((256, 1), lambda i: (i, 0)),
        out_shape=jax.ShapeDtypeStruct((_NPAD, 1), jnp.int32),
    )(s_col, s_row)

    data = jnp.concatenate(
        [boxes.astype(f32), scores.astype(f32)[:, None],
         jnp.zeros((_N, 3), f32)], axis=1)
    data = jnp.concatenate([data, jnp.zeros((_NPAD - _N, 8), f32)], axis=0)
    rank_row = ranks.reshape(1, _NPAD)

    top = pl.pallas_call(
        _select_kernel,
        grid=(_K // 512,),
        in_specs=[
            pl.BlockSpec((1, _NPAD), lambda i: (0, 0)),
            pl.BlockSpec((_NPAD, 8), lambda i: (0, 0)),
        ],
        out_specs=pl.BlockSpec((512, 8), lambda i: (i, 0)),
        out_shape=jax.ShapeDtypeStruct((_K, 8), f32),
    )(rank_row, data)

    out = pl.pallas_call(
        _nms_kernel,
        in_specs=[
            pl.BlockSpec((_K, 8), lambda: (0, 0)),
            pl.BlockSpec((8, _K), lambda: (0, 0)),
        ],
        out_specs=pl.BlockSpec((_K, 8), lambda: (0, 0)),
        out_shape=jax.ShapeDtypeStruct((_K, 8), f32),
        scratch_shapes=[pltpu.VMEM((_K, _K), jnp.int8),
                        pltpu.VMEM((_K, 1), jnp.float32),
                        pltpu.VMEM((1, _K), jnp.float32)],
    )(top, top.T)
    return out[:, :5]


# speedup vs baseline: 24.4730x; 1.1273x over previous
import functools

import jax
import jax.numpy as jnp
from jax.experimental import pallas as pl
from jax.experimental.pallas import tpu as pltpu

_N = 20000
_NPAD = 20480
_K = 4096
_THR = 0.5
_JCH = 2048
_CH = 256


def _sort_key(x):
    b = jax.lax.bitcast_convert_type(x, jnp.int32)
    return jnp.where(b >= 0, b, b ^ jnp.int32(0x7FFFFFFF))


def _key_kernel(s_row_ref, k_row_ref):
    k_row_ref[:, :] = _sort_key(s_row_ref[:, :])


def _rank_kernel(s_col_ref, k_row_ref, rank_ref):
    nblk_per_chunk = _JCH // 256
    i0 = pl.program_id(0) * 256
    ki = _sort_key(s_col_ref[:, :])
    ii = i0 + jax.lax.broadcasted_iota(jnp.int32, (256, 1), 0)
    cblk = pl.program_id(0) // nblk_per_chunk
    acc = jnp.zeros((256, 128), jnp.float32)
    for c in range(_NPAD // _JCH):
        kj = k_row_ref[:, c * _JCH:(c + 1) * _JCH]
        thr = jnp.where(c < cblk, ki - 1, ki)
        b = (kj > thr).astype(jnp.float32)
        for t in range(_JCH // 128):
            acc = acc + b[:, t * 128:(t + 1) * 128]
    rank_ref[:, :] = jnp.sum(acc, axis=1, keepdims=True).astype(jnp.int32)
    for c in range(_NPAD // _JCH):
        @pl.when(c == cblk)
        def _(c=c):
            kj = k_row_ref[:, c * _JCH:(c + 1) * _JCH]
            jj = c * _JCH + jax.lax.broadcasted_iota(jnp.int32, (1, _JCH), 1)
            tie = (kj == ki) & (jj < ii)
            corr = jnp.sum(tie.astype(jnp.float32), axis=1, keepdims=True)
            rank_ref[:, :] = rank_ref[:, :] + corr.astype(jnp.int32)


def _select_kernel(rank_row_ref, data_ref, out_ref):
    bf16 = jnp.bfloat16
    f32 = jnp.float32
    p0 = pl.program_id(0) * 512
    pp = p0 + jax.lax.broadcasted_iota(jnp.int32, (512, 1), 0)
    acc = jnp.zeros((512, 24), jnp.float32)
    for c in range(_NPAD // _JCH):
        rr = rank_row_ref[:, c * _JCH:(c + 1) * _JCH]
        oh = (rr == pp).astype(bf16)
        d = data_ref[c * _JCH:(c + 1) * _JCH, :]
        d_hi = d.astype(bf16)
        r1 = d - d_hi.astype(f32)
        d_mid = r1.astype(bf16)
        r2 = r1 - d_mid.astype(f32)
        d_lo = r2.astype(bf16)
        dcat = jnp.concatenate([d_hi, d_mid, d_lo], axis=1)
        acc = acc + jax.lax.dot_general(
            oh, dcat, (((1,), (0,)), ((), ())),
            preferred_element_type=jnp.float32)
    out_ref[:, :] = acc[:, 0:8] + acc[:, 8:16] + acc[:, 16:24]


_B = 512


def _nms_kernel(top_ref, top_t_ref, out_ref, s_scr, kc_scr, kr_scr):
    x1r = top_t_ref[0:1, :]
    y1r = top_t_ref[1:2, :]
    x2r = top_t_ref[2:3, :]
    y2r = top_t_ref[3:4, :]
    area_r = (x2r - x1r) * (y2r - y1r)

    def s_body(ci, _):
        sl = pl.ds(ci * _CH, _CH)
        x1c = top_ref[sl, 0:1]
        y1c = top_ref[sl, 1:2]
        x2c = top_ref[sl, 2:3]
        y2c = top_ref[sl, 3:4]
        ix1 = jnp.maximum(x1c, x1r)
        iy1 = jnp.maximum(y1c, y1r)
        ix2 = jnp.minimum(x2c, x2r)
        iy2 = jnp.minimum(y2c, y2r)
        iw = jnp.maximum(ix2 - ix1, 0.0)
        ih = jnp.maximum(iy2 - iy1, 0.0)
        inter = iw * ih
        union = (x2c - x1c) * (y2c - y1c) + area_r - inter
        iou = inter / jnp.maximum(union, 1e-9)
        s_scr[sl, :] = (iou > _THR).astype(jnp.int8)
        return 0

    jax.lax.fori_loop(0, _K // _CH, s_body, 0)
    kr_scr[:, :] = jnp.ones((1, _K), jnp.float32)

    rr = jax.lax.broadcasted_iota(jnp.int32, (_B, _B), 0)
    cc = jax.lax.broadcasted_iota(jnp.int32, (_B, _B), 1)
    ident = (rr == cc).astype(jnp.float32)
    tri_up = rr < cc
    tri_lo = cc < rr
    jlane = jax.lax.broadcasted_iota(jnp.int32, (1, _K), 1)

    for b in range(_K // _B):
        lo = b * _B
        hi = (b + 1) * _B
        sbb = s_scr[lo:hi, lo:hi] != 0
        up_m = sbb & tri_up
        lo_m = sbb & tri_lo
        ext_row = kr_scr[0:1, lo:hi]
        ext_col = jnp.sum(ident * ext_row, axis=1, keepdims=True)

        def cond(st):
            _, done, it = st
            return jnp.logical_and(jnp.logical_not(done), it < _B)

        def body(st, ext_row=ext_row, ext_col=ext_col, up_m=up_m, lo_m=lo_m):
            kcol, _, it = st
            hit_r = jnp.max((up_m & (kcol > 0.0)).astype(jnp.float32),
                            axis=0, keepdims=True)
            krow = ext_row * (1.0 - hit_r)
            hit_c = jnp.max((lo_m & (krow > 0.0)).astype(jnp.float32),
                            axis=1, keepdims=True)
            kcol2 = ext_col * (1.0 - hit_c)
            done = jnp.all(kcol2 == kcol)
            return kcol2, done, it + 1

        kcol, _, _ = jax.lax.while_loop(
            cond, body, (ext_col, False, jnp.int32(0)))
        kc_scr[lo:hi, :] = kcol
        if b + 1 < _K // _B:
            srow = s_scr[lo:hi, :] != 0
            supp = jnp.max((srow & (kcol > 0.0)).astype(jnp.float32),
                           axis=0, keepdims=True)
            later = (jlane >= hi).astype(jnp.float32)
            kr_scr[0:1, :] = kr_scr[0:1, :] * (1.0 - supp * later)

    out_ref[:, :] = top_ref[:, :] * kc_scr[:, :]


@jax.jit
def kernel(boxes, scores):
    f32 = jnp.float32
    s_pad = jnp.concatenate(
        [scores.astype(f32), jnp.full((_NPAD - _N,), -1.0, f32)])
    s_col = s_pad.reshape(_NPAD, 1)
    s_row = s_pad.reshape(1, _NPAD)

    k_row = pl.pallas_call(
        _key_kernel,
        out_shape=jax.ShapeDtypeStruct((1, _NPAD), jnp.int32),
    )(s_row)

    ranks = pl.pallas_call(
        _rank_kernel,
        grid=(_NPAD // 256,),
        in_specs=[
            pl.BlockSpec((256, 1), lambda i: (i, 0)),
            pl.BlockSpec((1, _NPAD), lambda i: (0, 0)),
        ],
        out_specs=pl.BlockSpec((256, 1), lambda i: (i, 0)),
        out_shape=jax.ShapeDtypeStruct((_NPAD, 1), jnp.int32),
    )(s_col, k_row)

    data = jnp.concatenate(
        [boxes.astype(f32), scores.astype(f32)[:, None],
         jnp.zeros((_N, 3), f32)], axis=1)
    data = jnp.concatenate([data, jnp.zeros((_NPAD - _N, 8), f32)], axis=0)
    rank_row = ranks.reshape(1, _NPAD)

    top = pl.pallas_call(
        _select_kernel,
        grid=(_K // 512,),
        in_specs=[
            pl.BlockSpec((1, _NPAD), lambda i: (0, 0)),
            pl.BlockSpec((_NPAD, 8), lambda i: (0, 0)),
        ],
        out_specs=pl.BlockSpec((512, 8), lambda i: (i, 0)),
        out_shape=jax.ShapeDtypeStruct((_K, 8), f32),
    )(rank_row, data)

    out = pl.pallas_call(
        _nms_kernel,
        in_specs=[
            pl.BlockSpec((_K, 8), lambda: (0, 0)),
            pl.BlockSpec((8, _K), lambda: (0, 0)),
        ],
        out_specs=pl.BlockSpec((_K, 8), lambda: (0, 0)),
        out_shape=jax.ShapeDtypeStruct((_K, 8), f32),
        scratch_shapes=[pltpu.VMEM((_K, _K), jnp.int8),
                        pltpu.VMEM((_K, 1), jnp.float32),
                        pltpu.VMEM((1, _K), jnp.float32)],
    )(top, top.T)
    return out[:, :5]


# speedup vs baseline: 25.4880x; 1.0415x over previous
import functools

import jax
import jax.numpy as jnp
from jax.experimental import pallas as pl
from jax.experimental.pallas import tpu as pltpu

_N = 20000
_NPAD = 20480
_K = 4096
_THR = 0.5
_JCH = 2048
_RB = 512
_CH = 256


def _sort_key(x):
    b = jax.lax.bitcast_convert_type(x, jnp.int32)
    return jnp.where(b >= 0, b, b ^ jnp.int32(0x7FFFFFFF))


def _key_kernel(s_row_ref, k_row_ref):
    k_row_ref[:, :] = _sort_key(s_row_ref[:, :])


def _rank_kernel(s_col_ref, k_row_ref, rank_ref):
    nblk_per_chunk = _JCH // _RB
    i0 = pl.program_id(0) * _RB
    ki = _sort_key(s_col_ref[:, :])
    ii = i0 + jax.lax.broadcasted_iota(jnp.int32, (_RB, 1), 0)
    cblk = pl.program_id(0) // nblk_per_chunk
    acc = jnp.zeros((_RB, 128), jnp.float32)
    for c in range(_NPAD // _JCH):
        kj = k_row_ref[:, c * _JCH:(c + 1) * _JCH]
        thr = jnp.where(c < cblk, ki - 1, ki)
        b = (kj > thr).astype(jnp.float32)
        for t in range(_JCH // 128):
            acc = acc + b[:, t * 128:(t + 1) * 128]
    rank_ref[:, :] = jnp.sum(acc, axis=1, keepdims=True).astype(jnp.int32)
    for c in range(_NPAD // _JCH):
        @pl.when(c == cblk)
        def _(c=c):
            kj = k_row_ref[:, c * _JCH:(c + 1) * _JCH]
            jj = c * _JCH + jax.lax.broadcasted_iota(jnp.int32, (1, _JCH), 1)
            tie = (kj == ki) & (jj < ii)
            corr = jnp.sum(tie.astype(jnp.float32), axis=1, keepdims=True)
            rank_ref[:, :] = rank_ref[:, :] + corr.astype(jnp.int32)


def _select_kernel(rank_row_ref, data_ref, out_ref):
    bf16 = jnp.bfloat16
    f32 = jnp.float32
    p0 = pl.program_id(0) * 512
    pp = p0 + jax.lax.broadcasted_iota(jnp.int32, (512, 1), 0)
    acc = jnp.zeros((512, 24), jnp.float32)
    for c in range(_NPAD // _JCH):
        rr = rank_row_ref[:, c * _JCH:(c + 1) * _JCH]
        oh = (rr == pp).astype(bf16)
        d = data_ref[c * _JCH:(c + 1) * _JCH, :]
        d_hi = d.astype(bf16)
        r1 = d - d_hi.astype(f32)
        d_mid = r1.astype(bf16)
        r2 = r1 - d_mid.astype(f32)
        d_lo = r2.astype(bf16)
        dcat = jnp.concatenate([d_hi, d_mid, d_lo], axis=1)
        acc = acc + jax.lax.dot_general(
            oh, dcat, (((1,), (0,)), ((), ())),
            preferred_element_type=jnp.float32)
    out_ref[:, :] = acc[:, 0:8] + acc[:, 8:16] + acc[:, 16:24]


_B = 512


def _nms_kernel(top_ref, top_t_ref, out_ref, s_scr, kc_scr, kr_scr):
    x1r = top_t_ref[0:1, :]
    y1r = top_t_ref[1:2, :]
    x2r = top_t_ref[2:3, :]
    y2r = top_t_ref[3:4, :]
    area_r = (x2r - x1r) * (y2r - y1r)

    def s_body(ci, _):
        sl = pl.ds(ci * _CH, _CH)
        x1c = top_ref[sl, 0:1]
        y1c = top_ref[sl, 1:2]
        x2c = top_ref[sl, 2:3]
        y2c = top_ref[sl, 3:4]
        ix1 = jnp.maximum(x1c, x1r)
        iy1 = jnp.maximum(y1c, y1r)
        ix2 = jnp.minimum(x2c, x2r)
        iy2 = jnp.minimum(y2c, y2r)
        iw = jnp.maximum(ix2 - ix1, 0.0)
        ih = jnp.maximum(iy2 - iy1, 0.0)
        inter = iw * ih
        union = (x2c - x1c) * (y2c - y1c) + area_r - inter
        iou = inter / jnp.maximum(union, 1e-9)
        s_scr[sl, :] = (iou > _THR).astype(jnp.int8)
        return 0

    jax.lax.fori_loop(0, _K // _CH, s_body, 0)
    kr_scr[:, :] = jnp.ones((1, _K), jnp.float32)

    rr = jax.lax.broadcasted_iota(jnp.int32, (_B, _B), 0)
    cc = jax.lax.broadcasted_iota(jnp.int32, (_B, _B), 1)
    ident = (rr == cc).astype(jnp.float32)
    tri_up = rr < cc
    tri_lo = cc < rr
    jlane = jax.lax.broadcasted_iota(jnp.int32, (1, _K), 1)

    for b in range(_K // _B):
        lo = b * _B
        hi = (b + 1) * _B
        sbb = s_scr[lo:hi, lo:hi] != 0
        up_m = sbb & tri_up
        lo_m = sbb & tri_lo
        ext_row = kr_scr[0:1, lo:hi]
        ext_col = jnp.sum(ident * ext_row, axis=1, keepdims=True)

        def cond(st):
            _, done, it = st
            return jnp.logical_and(jnp.logical_not(done), it < _B)

        def body(st, ext_row=ext_row, ext_col=ext_col, up_m=up_m, lo_m=lo_m):
            kcol, _, it = st
            hit_r = jnp.max((up_m & (kcol > 0.0)).astype(jnp.float32),
                            axis=0, keepdims=True)
            krow = ext_row * (1.0 - hit_r)
            hit_c = jnp.max((lo_m & (krow > 0.0)).astype(jnp.float32),
                            axis=1, keepdims=True)
            kcol2 = ext_col * (1.0 - hit_c)
            done = jnp.all(kcol2 == kcol)
            return kcol2, done, it + 1

        kcol, _, _ = jax.lax.while_loop(
            cond, body, (ext_col, False, jnp.int32(0)))
        kc_scr[lo:hi, :] = kcol
        if b + 1 < _K // _B:
            srow = s_scr[lo:hi, :] != 0
            supp = jnp.max((srow & (kcol > 0.0)).astype(jnp.float32),
                           axis=0, keepdims=True)
            later = (jlane >= hi).astype(jnp.float32)
            kr_scr[0:1, :] = kr_scr[0:1, :] * (1.0 - supp * later)

    out_ref[:, :] = top_ref[:, :] * kc_scr[:, :]


@jax.jit
def kernel(boxes, scores):
    f32 = jnp.float32
    s_pad = jnp.concatenate(
        [scores.astype(f32), jnp.full((_NPAD - _N,), -1.0, f32)])
    s_col = s_pad.reshape(_NPAD, 1)
    s_row = s_pad.reshape(1, _NPAD)

    k_row = pl.pallas_call(
        _key_kernel,
        out_shape=jax.ShapeDtypeStruct((1, _NPAD), jnp.int32),
    )(s_row)

    ranks = pl.pallas_call(
        _rank_kernel,
        grid=(_NPAD // _RB,),
        in_specs=[
            pl.BlockSpec((_RB, 1), lambda i: (i, 0)),
            pl.BlockSpec((1, _NPAD), lambda i: (0, 0)),
        ],
        out_specs=pl.BlockSpec((_RB, 1), lambda i: (i, 0)),
        out_shape=jax.ShapeDtypeStruct((_NPAD, 1), jnp.int32),
    )(s_col, k_row)

    data = jnp.concatenate(
        [boxes.astype(f32), scores.astype(f32)[:, None],
         jnp.zeros((_N, 3), f32)], axis=1)
    data = jnp.concatenate([data, jnp.zeros((_NPAD - _N, 8), f32)], axis=0)
    rank_row = ranks.reshape(1, _NPAD)

    top = pl.pallas_call(
        _select_kernel,
        grid=(_K // 512,),
        in_specs=[
            pl.BlockSpec((1, _NPAD), lambda i: (0, 0)),
            pl.BlockSpec((_NPAD, 8), lambda i: (0, 0)),
        ],
        out_specs=pl.BlockSpec((512, 8), lambda i: (i, 0)),
        out_shape=jax.ShapeDtypeStruct((_K, 8), f32),
    )(rank_row, data)

    out = pl.pallas_call(
        _nms_kernel,
        in_specs=[
            pl.BlockSpec((_K, 8), lambda: (0, 0)),
            pl.BlockSpec((8, _K), lambda: (0, 0)),
        ],
        out_specs=pl.BlockSpec((_K, 8), lambda: (0, 0)),
        out_shape=jax.ShapeDtypeStruct((_K, 8), f32),
        scratch_shapes=[pltpu.VMEM((_K, _K), jnp.int8),
                        pltpu.VMEM((_K, 1), jnp.float32),
                        pltpu.VMEM((1, _K), jnp.float32)],
    )(top, top.T)
    return out[:, :5]


# speedup vs baseline: 25.7920x; 1.0119x over previous
import functools

import jax
import jax.numpy as jnp
from jax.experimental import pallas as pl
from jax.experimental.pallas import tpu as pltpu

_N = 20000
_NPAD = 20480
_K = 4096
_THR = 0.5
_JCH = 2048
_RB = 512
_CH = 256


def _sort_key(x):
    b = jax.lax.bitcast_convert_type(x, jnp.int32)
    return jnp.where(b >= 0, b, b ^ jnp.int32(0x7FFFFFFF))


def _key_kernel(s_row_ref, k_row_ref):
    k_row_ref[:, :] = _sort_key(s_row_ref[:, :])


def _rank_kernel(s_col_ref, k_row_ref, rank_ref):
    nblk_per_chunk = _JCH // _RB
    i0 = pl.program_id(0) * _RB
    ki = _sort_key(s_col_ref[:, :])
    ii = i0 + jax.lax.broadcasted_iota(jnp.int32, (_RB, 1), 0)
    cblk = pl.program_id(0) // nblk_per_chunk
    acc = jnp.zeros((_RB, 128), jnp.float32)
    for c in range(_NPAD // _JCH):
        kj = k_row_ref[:, c * _JCH:(c + 1) * _JCH]
        thr = jnp.where(c < cblk, ki - 1, ki)
        b = (kj > thr).astype(jnp.float32)
        for t in range(_JCH // 128):
            acc = acc + b[:, t * 128:(t + 1) * 128]
    rank_ref[:, :] = jnp.sum(acc, axis=1, keepdims=True).astype(jnp.int32)
    for c in range(_NPAD // _JCH):
        @pl.when(c == cblk)
        def _(c=c):
            kj = k_row_ref[:, c * _JCH:(c + 1) * _JCH]
            jj = c * _JCH + jax.lax.broadcasted_iota(jnp.int32, (1, _JCH), 1)
            tie = (kj == ki) & (jj < ii)
            corr = jnp.sum(tie.astype(jnp.float32), axis=1, keepdims=True)
            rank_ref[:, :] = rank_ref[:, :] + corr.astype(jnp.int32)


def _select_kernel(rank_row_ref, data_ref, out_ref):
    bf16 = jnp.bfloat16
    f32 = jnp.float32
    p0 = pl.program_id(0) * 512
    pp = p0 + jax.lax.broadcasted_iota(jnp.int32, (512, 1), 0)
    acc = jnp.zeros((512, 24), jnp.float32)
    for c in range(_NPAD // _JCH):
        rr = rank_row_ref[:, c * _JCH:(c + 1) * _JCH]
        oh = (rr == pp).astype(bf16)
        d = data_ref[c * _JCH:(c + 1) * _JCH, :]
        d_hi = d.astype(bf16)
        r1 = d - d_hi.astype(f32)
        d_mid = r1.astype(bf16)
        r2 = r1 - d_mid.astype(f32)
        d_lo = r2.astype(bf16)
        dcat = jnp.concatenate([d_hi, d_mid, d_lo], axis=1)
        acc = acc + jax.lax.dot_general(
            oh, dcat, (((1,), (0,)), ((), ())),
            preferred_element_type=jnp.float32)
    out_ref[:, :] = acc[:, 0:8] + acc[:, 8:16] + acc[:, 16:24]


_B = 512


def _nms_kernel(top_ref, top_t_ref, out_ref, s_scr, kc_scr, kr_scr):
    x1r = top_t_ref[0:1, :]
    y1r = top_t_ref[1:2, :]
    x2r = top_t_ref[2:3, :]
    y2r = top_t_ref[3:4, :]
    area_r = (x2r - x1r) * (y2r - y1r)

    def s_body(ci, _):
        sl = pl.ds(ci * _CH, _CH)
        x1c = top_ref[sl, 0:1]
        y1c = top_ref[sl, 1:2]
        x2c = top_ref[sl, 2:3]
        y2c = top_ref[sl, 3:4]
        ix1 = jnp.maximum(x1c, x1r)
        iy1 = jnp.maximum(y1c, y1r)
        ix2 = jnp.minimum(x2c, x2r)
        iy2 = jnp.minimum(y2c, y2r)
        iw = jnp.maximum(ix2 - ix1, 0.0)
        ih = jnp.maximum(iy2 - iy1, 0.0)
        inter = iw * ih
        union = (x2c - x1c) * (y2c - y1c) + area_r - inter
        iou = inter / jnp.maximum(union, 1e-9)
        s_scr[sl, :] = (iou > _THR).astype(jnp.int8)
        return 0

    jax.lax.fori_loop(0, _K // _CH, s_body, 0)
    kr_scr[:, :] = jnp.ones((1, _K), jnp.float32)

    rr = jax.lax.broadcasted_iota(jnp.int32, (_B, _B), 0)
    cc = jax.lax.broadcasted_iota(jnp.int32, (_B, _B), 1)
    ident = (rr == cc).astype(jnp.float32)
    tri_up = rr < cc
    tri_lo = cc < rr
    jlane = jax.lax.broadcasted_iota(jnp.int32, (1, _K), 1)

    for b in range(_K // _B):
        lo = b * _B
        hi = (b + 1) * _B
        sbb = s_scr[lo:hi, lo:hi] != 0
        up_m = sbb & tri_up
        lo_m = sbb & tri_lo
        ext_row = kr_scr[0:1, lo:hi]
        ext_col = jnp.sum(ident * ext_row, axis=1, keepdims=True)

        def cond(st):
            _, done, it = st
            return jnp.logical_and(jnp.logical_not(done), it < _B)

        def tt(kcol, ext_row=ext_row, ext_col=ext_col, up_m=up_m, lo_m=lo_m):
            hit_r = jnp.max((up_m & (kcol > 0.0)).astype(jnp.float32),
                            axis=0, keepdims=True)
            krow = ext_row * (1.0 - hit_r)
            hit_c = jnp.max((lo_m & (krow > 0.0)).astype(jnp.float32),
                            axis=1, keepdims=True)
            return ext_col * (1.0 - hit_c)

        def body(st):
            kcol, _, it = st
            kcol_a = tt(kcol)
            kcol_b = tt(kcol_a)
            done = jnp.all(kcol_b == kcol_a)
            return kcol_b, done, it + 1

        kcol, _, _ = jax.lax.while_loop(
            cond, body, (ext_col, False, jnp.int32(0)))
        kc_scr[lo:hi, :] = kcol
        if b + 1 < _K // _B:
            srow = s_scr[lo:hi, :] != 0
            supp = jnp.max((srow & (kcol > 0.0)).astype(jnp.float32),
                           axis=0, keepdims=True)
            later = (jlane >= hi).astype(jnp.float32)
            kr_scr[0:1, :] = kr_scr[0:1, :] * (1.0 - supp * later)

    out_ref[:, :] = top_ref[:, :] * kc_scr[:, :]


@jax.jit
def kernel(boxes, scores):
    f32 = jnp.float32
    s_pad = jnp.concatenate(
        [scores.astype(f32), jnp.full((_NPAD - _N,), -1.0, f32)])
    s_col = s_pad.reshape(_NPAD, 1)
    s_row = s_pad.reshape(1, _NPAD)

    k_row = pl.pallas_call(
        _key_kernel,
        out_shape=jax.ShapeDtypeStruct((1, _NPAD), jnp.int32),
    )(s_row)

    ranks = pl.pallas_call(
        _rank_kernel,
        grid=(_NPAD // _RB,),
        in_specs=[
            pl.BlockSpec((_RB, 1), lambda i: (i, 0)),
            pl.BlockSpec((1, _NPAD), lambda i: (0, 0)),
        ],
        out_specs=pl.BlockSpec((_RB, 1), lambda i: (i, 0)),
        out_shape=jax.ShapeDtypeStruct((_NPAD, 1), jnp.int32),
    )(s_col, k_row)

    data = jnp.concatenate(
        [boxes.astype(f32), scores.astype(f32)[:, None],
         jnp.zeros((_N, 3), f32)], axis=1)
    data = jnp.concatenate([data, jnp.zeros((_NPAD - _N, 8), f32)], axis=0)
    rank_row = ranks.reshape(1, _NPAD)

    top = pl.pallas_call(
        _select_kernel,
        grid=(_K // 512,),
        in_specs=[
            pl.BlockSpec((1, _NPAD), lambda i: (0, 0)),
            pl.BlockSpec((_NPAD, 8), lambda i: (0, 0)),
        ],
        out_specs=pl.BlockSpec((512, 8), lambda i: (i, 0)),
        out_shape=jax.ShapeDtypeStruct((_K, 8), f32),
    )(rank_row, data)

    out = pl.pallas_call(
        _nms_kernel,
        in_specs=[
            pl.BlockSpec((_K, 8), lambda: (0, 0)),
            pl.BlockSpec((8, _K), lambda: (0, 0)),
        ],
        out_specs=pl.BlockSpec((_K, 8), lambda: (0, 0)),
        out_shape=jax.ShapeDtypeStruct((_K, 8), f32),
        scratch_shapes=[pltpu.VMEM((_K, _K), jnp.int8),
                        pltpu.VMEM((_K, 1), jnp.float32),
                        pltpu.VMEM((1, _K), jnp.float32)],
    )(top, top.T)
    return out[:, :5]


# speedup vs baseline: 26.0587x; 1.0103x over previous
import functools

import jax
import jax.numpy as jnp
from jax.experimental import pallas as pl
from jax.experimental.pallas import tpu as pltpu

_N = 20000
_NPAD = 20480
_K = 4096
_THR = 0.5
_JCH = 2048
_RB = 512
_CH = 512


def _sort_key(x):
    b = jax.lax.bitcast_convert_type(x, jnp.int32)
    return jnp.where(b >= 0, b, b ^ jnp.int32(0x7FFFFFFF))


def _key_kernel(s_row_ref, k_row_ref):
    k_row_ref[:, :] = _sort_key(s_row_ref[:, :])


def _rank_kernel(s_col_ref, k_row_ref, rank_ref):
    nblk_per_chunk = _JCH // _RB
    i0 = pl.program_id(0) * _RB
    ki = _sort_key(s_col_ref[:, :])
    ii = i0 + jax.lax.broadcasted_iota(jnp.int32, (_RB, 1), 0)
    cblk = pl.program_id(0) // nblk_per_chunk
    acc = jnp.zeros((_RB, 128), jnp.float32)
    for c in range(_NPAD // _JCH):
        kj = k_row_ref[:, c * _JCH:(c + 1) * _JCH]
        thr = jnp.where(c < cblk, ki - 1, ki)
        b = (kj > thr).astype(jnp.float32)
        for t in range(_JCH // 128):
            acc = acc + b[:, t * 128:(t + 1) * 128]
    rank_ref[:, :] = jnp.sum(acc, axis=1, keepdims=True).astype(jnp.int32)
    for c in range(_NPAD // _JCH):
        @pl.when(c == cblk)
        def _(c=c):
            kj = k_row_ref[:, c * _JCH:(c + 1) * _JCH]
            jj = c * _JCH + jax.lax.broadcasted_iota(jnp.int32, (1, _JCH), 1)
            tie = (kj == ki) & (jj < ii)
            corr = jnp.sum(tie.astype(jnp.float32), axis=1, keepdims=True)
            rank_ref[:, :] = rank_ref[:, :] + corr.astype(jnp.int32)


def _select_kernel(rank_row_ref, data_ref, out_ref):
    bf16 = jnp.bfloat16
    f32 = jnp.float32
    p0 = pl.program_id(0) * 512
    pp = p0 + jax.lax.broadcasted_iota(jnp.int32, (512, 1), 0)
    acc = jnp.zeros((512, 24), jnp.float32)
    for c in range(_NPAD // _JCH):
        rr = rank_row_ref[:, c * _JCH:(c + 1) * _JCH]
        oh = (rr == pp).astype(bf16)
        d = data_ref[c * _JCH:(c + 1) * _JCH, :]
        d_hi = d.astype(bf16)
        r1 = d - d_hi.astype(f32)
        d_mid = r1.astype(bf16)
        r2 = r1 - d_mid.astype(f32)
        d_lo = r2.astype(bf16)
        dcat = jnp.concatenate([d_hi, d_mid, d_lo], axis=1)
        acc = acc + jax.lax.dot_general(
            oh, dcat, (((1,), (0,)), ((), ())),
            preferred_element_type=jnp.float32)
    out_ref[:, :] = acc[:, 0:8] + acc[:, 8:16] + acc[:, 16:24]


_B = 512


def _nms_kernel(top_ref, top_t_ref, out_ref, s_scr, kc_scr, kr_scr):
    x1r = top_t_ref[0:1, :]
    y1r = top_t_ref[1:2, :]
    x2r = top_t_ref[2:3, :]
    y2r = top_t_ref[3:4, :]
    area_r = (x2r - x1r) * (y2r - y1r)

    def s_body(ci, _):
        sl = pl.ds(ci * _CH, _CH)
        x1c = top_ref[sl, 0:1]
        y1c = top_ref[sl, 1:2]
        x2c = top_ref[sl, 2:3]
        y2c = top_ref[sl, 3:4]
        ix1 = jnp.maximum(x1c, x1r)
        iy1 = jnp.maximum(y1c, y1r)
        ix2 = jnp.minimum(x2c, x2r)
        iy2 = jnp.minimum(y2c, y2r)
        iw = jnp.maximum(ix2 - ix1, 0.0)
        ih = jnp.maximum(iy2 - iy1, 0.0)
        inter = iw * ih
        union = (x2c - x1c) * (y2c - y1c) + area_r - inter
        iou = inter / jnp.maximum(union, 1e-9)
        s_scr[sl, :] = (iou > _THR).astype(jnp.int8)
        return 0

    jax.lax.fori_loop(0, _K // _CH, s_body, 0)
    kr_scr[:, :] = jnp.ones((1, _K), jnp.float32)

    rr = jax.lax.broadcasted_iota(jnp.int32, (_B, _B), 0)
    cc = jax.lax.broadcasted_iota(jnp.int32, (_B, _B), 1)
    ident = (rr == cc).astype(jnp.float32)
    tri_up = rr < cc
    tri_lo = cc < rr
    jlane = jax.lax.broadcasted_iota(jnp.int32, (1, _K), 1)

    for b in range(_K // _B):
        lo = b * _B
        hi = (b + 1) * _B
        sbb = s_scr[lo:hi, lo:hi] != 0
        up_m = sbb & tri_up
        lo_m = sbb & tri_lo
        ext_row = kr_scr[0:1, lo:hi]
        ext_col = jnp.sum(ident * ext_row, axis=1, keepdims=True)

        def cond(st):
            _, done, it = st
            return jnp.logical_and(jnp.logical_not(done), it < _B)

        def tt(kcol, ext_row=ext_row, ext_col=ext_col, up_m=up_m, lo_m=lo_m):
            hit_r = jnp.max((up_m & (kcol > 0.0)).astype(jnp.float32),
                            axis=0, keepdims=True)
            krow = ext_row * (1.0 - hit_r)
            hit_c = jnp.max((lo_m & (krow > 0.0)).astype(jnp.float32),
                            axis=1, keepdims=True)
            return ext_col * (1.0 - hit_c)

        def body(st):
            kcol, _, it = st
            kcol_a = tt(kcol)
            kcol_b = tt(kcol_a)
            done = jnp.all(kcol_b == kcol_a)
            return kcol_b, done, it + 1

        kcol, _, _ = jax.lax.while_loop(
            cond, body, (ext_col, False, jnp.int32(0)))
        kc_scr[lo:hi, :] = kcol
        if b + 1 < _K // _B:
            srow = s_scr[lo:hi, :] != 0
            supp = jnp.max((srow & (kcol > 0.0)).astype(jnp.float32),
                           axis=0, keepdims=True)
            later = (jlane >= hi).astype(jnp.float32)
            kr_scr[0:1, :] = kr_scr[0:1, :] * (1.0 - supp * later)

    out_ref[:, :] = top_ref[:, :] * kc_scr[:, :]


@jax.jit
def kernel(boxes, scores):
    f32 = jnp.float32
    s_pad = jnp.concatenate(
        [scores.astype(f32), jnp.full((_NPAD - _N,), -1.0, f32)])
    s_col = s_pad.reshape(_NPAD, 1)
    s_row = s_pad.reshape(1, _NPAD)

    k_row = pl.pallas_call(
        _key_kernel,
        out_shape=jax.ShapeDtypeStruct((1, _NPAD), jnp.int32),
    )(s_row)

    ranks = pl.pallas_call(
        _rank_kernel,
        grid=(_NPAD // _RB,),
        in_specs=[
            pl.BlockSpec((_RB, 1), lambda i: (i, 0)),
            pl.BlockSpec((1, _NPAD), lambda i: (0, 0)),
        ],
        out_specs=pl.BlockSpec((_RB, 1), lambda i: (i, 0)),
        out_shape=jax.ShapeDtypeStruct((_NPAD, 1), jnp.int32),
    )(s_col, k_row)

    data = jnp.concatenate(
        [boxes.astype(f32), scores.astype(f32)[:, None],
         jnp.zeros((_N, 3), f32)], axis=1)
    data = jnp.concatenate([data, jnp.zeros((_NPAD - _N, 8), f32)], axis=0)
    rank_row = ranks.reshape(1, _NPAD)

    top = pl.pallas_call(
        _select_kernel,
        grid=(_K // 512,),
        in_specs=[
            pl.BlockSpec((1, _NPAD), lambda i: (0, 0)),
            pl.BlockSpec((_NPAD, 8), lambda i: (0, 0)),
        ],
        out_specs=pl.BlockSpec((512, 8), lambda i: (i, 0)),
        out_shape=jax.ShapeDtypeStruct((_K, 8), f32),
    )(rank_row, data)

    out = pl.pallas_call(
        _nms_kernel,
        in_specs=[
            pl.BlockSpec((_K, 8), lambda: (0, 0)),
            pl.BlockSpec((8, _K), lambda: (0, 0)),
        ],
        out_specs=pl.BlockSpec((_K, 8), lambda: (0, 0)),
        out_shape=jax.ShapeDtypeStruct((_K, 8), f32),
        scratch_shapes=[pltpu.VMEM((_K, _K), jnp.int8),
                        pltpu.VMEM((_K, 1), jnp.float32),
                        pltpu.VMEM((1, _K), jnp.float32)],
    )(top, top.T)
    return out[:, :5]
